# Initial kernel scaffold; baseline (speedup 1.0000x reference)
import jax
import jax.numpy as jnp
from jax.experimental import pallas as pl


def kernel(x, h, edge_fea, edge_index, ew1, eb1, ew2, eb2, cw1, cb1, cw2, cb2, nw1, nb1, nw2, nb2):
    raise NotImplementedError("write your pallas kernel here")


# speedup vs baseline: 3.3774x; 3.3774x over previous
import functools

import jax
import jax.numpy as jnp
from jax import lax
from jax.experimental import pallas as pl
from jax.experimental.pallas import tpu as pltpu
from jax.experimental.pallas import tpu_sc as plsc

_NC, _NS, _L = 2, 16, 16
_NW = _NC * _NS


def _premul_body(h_ref, wa_ref, wb_ref, a_ref, b_ref):
    h = h_ref[...]
    a_ref[...] = jnp.dot(h, wa_ref[...], preferred_element_type=jnp.float32)
    b_ref[...] = jnp.dot(h, wb_ref[...], preferred_element_type=jnp.float32)


def _make_sc_gather(N, E, H):
    ept = E // _NW
    K = 80
    n_chunks = ept // K
    mesh = plsc.VectorSubcoreMesh(
        core_axis_name="c", subcore_axis_name="s",
        num_cores=_NC, num_subcores=_NS)

    @functools.partial(
        pl.kernel,
        out_type=(
            jax.ShapeDtypeStruct((E, H), jnp.float32),
            jax.ShapeDtypeStruct((E, H), jnp.float32),
            jax.ShapeDtypeStruct((E, _L), jnp.float32),
            jax.ShapeDtypeStruct((E, _L), jnp.float32),
        ),
        mesh=mesh,
        scratch_types=[
            pltpu.VMEM((K,), jnp.int32),
            pltpu.VMEM((K,), jnp.int32),
            pltpu.VMEM((K, H), jnp.float32),
            pltpu.VMEM((K, H), jnp.float32),
            pltpu.VMEM((K, _L), jnp.float32),
            pltpu.VMEM((K, _L), jnp.float32),
            pltpu.SemaphoreType.DMA,
        ],
        compiler_params=pltpu.CompilerParams(use_tc_tiling_on_sc=False),
    )
    def gather_k(a_hbm, b_hbm, xp_hbm, row_hbm, col_hbm,
                 hr_hbm, hc_hbm, xr_hbm, xc_hbm,
                 idx_r, idx_c, buf_a, buf_b, buf_xr, buf_xc, sem):
        wid = lax.axis_index("c") * _NS + lax.axis_index("s")
        base0 = wid * ept

        def chunk(i, carry):
            base = base0 + i * K
            pltpu.sync_copy(row_hbm.at[pl.ds(base, K)], idx_r)
            pltpu.sync_copy(col_hbm.at[pl.ds(base, K)], idx_c)
            c1 = pltpu.async_copy(a_hbm.at[idx_r], buf_a, sem)
            c2 = pltpu.async_copy(b_hbm.at[idx_c], buf_b, sem)
            c3 = pltpu.async_copy(xp_hbm.at[idx_r], buf_xr, sem)
            c4 = pltpu.async_copy(xp_hbm.at[idx_c], buf_xc, sem)
            c1.wait(); c2.wait(); c3.wait(); c4.wait()
            pltpu.sync_copy(buf_a, hr_hbm.at[pl.ds(base, K)])
            pltpu.sync_copy(buf_b, hc_hbm.at[pl.ds(base, K)])
            pltpu.sync_copy(buf_xr, xr_hbm.at[pl.ds(base, K)])
            pltpu.sync_copy(buf_xc, xc_hbm.at[pl.ds(base, K)])
            return carry

        lax.fori_loop(0, n_chunks, chunk, 0)

    return gather_k


def _edge_body(hr_ref, hc_ref, xr_ref, xc_ref, ef_ref,
               w0_ref, wf_ref, eb1_ref, ew2_ref, eb2_ref,
               cw1_ref, cb1_ref, cw2_ref, cb2_ref,
               msg_ref, fr_ref):
    rij = xr_ref[...] - xc_ref[...]
    sq = jnp.sum(rij * rij, axis=-1, keepdims=True)
    pre = hr_ref[...] + hc_ref[...]
    pre = pre + sq * w0_ref[...]
    pre = pre + jnp.dot(ef_ref[...], wf_ref[...],
                        preferred_element_type=jnp.float32)
    m = jax.nn.silu(pre + eb1_ref[...])
    msg = jax.nn.silu(jnp.dot(m, ew2_ref[...],
                              preferred_element_type=jnp.float32) + eb2_ref[...])
    hid = jax.nn.silu(jnp.dot(msg, cw1_ref[...],
                              preferred_element_type=jnp.float32) + cb1_ref[...])
    cm = jnp.dot(hid, cw2_ref[...],
                 preferred_element_type=jnp.float32) + cb2_ref[...]
    msg_ref[...] = msg
    lane = lax.broadcasted_iota(jnp.int32, rij.shape, 1)
    fr_ref[...] = rij * cm + jnp.where(lane == 3, 1.0, 0.0).astype(jnp.float32)


def _make_sc_scatter(N, E, H):
    ept = E // _NW
    K = 80
    n_chunks = ept // K
    rows_pt = N // _NS
    mesh = plsc.VectorSubcoreMesh(
        core_axis_name="c", subcore_axis_name="s",
        num_cores=_NC, num_subcores=_NS)

    @functools.partial(
        pl.kernel,
        out_type=(
            jax.ShapeDtypeStruct((_NC, N, H), jnp.float32),
            jax.ShapeDtypeStruct((_NC, N, _L), jnp.float32),
        ),
        mesh=mesh,
        scratch_types=[
            pltpu.VMEM((K,), jnp.int32),
            pltpu.VMEM((K, H), jnp.float32),
            pltpu.VMEM((K, _L), jnp.float32),
            pltpu.VMEM_SHARED((N, H), jnp.float32),
            pltpu.VMEM_SHARED((N, _L), jnp.float32),
        ],
        compiler_params=pltpu.CompilerParams(use_tc_tiling_on_sc=False),
    )
    def scatter_k(msg_hbm, fr_hbm, row_hbm, z_m_hbm, z_f_hbm,
                  mp_hbm, fp_hbm,
                  idx_v, buf_m, buf_f, acc_m, acc_f):
        cid = lax.axis_index("c")
        sid = lax.axis_index("s")
        wid = cid * _NS + sid
        r0 = sid * rows_pt
        pltpu.sync_copy(z_m_hbm.at[pl.ds(r0, rows_pt)],
                        acc_m.at[pl.ds(r0, rows_pt)])
        pltpu.sync_copy(z_f_hbm.at[pl.ds(r0, rows_pt)],
                        acc_f.at[pl.ds(r0, rows_pt)])
        plsc.subcore_barrier()
        base0 = wid * ept

        def chunk(i, carry):
            base = base0 + i * K
            pltpu.sync_copy(row_hbm.at[pl.ds(base, K)], idx_v)
            pltpu.sync_copy(msg_hbm.at[pl.ds(base, K)], buf_m)
            pltpu.sync_copy(fr_hbm.at[pl.ds(base, K)], buf_f)
            pltpu.sync_copy(buf_m, acc_m.at[idx_v], add=True)
            pltpu.sync_copy(buf_f, acc_f.at[idx_v], add=True)
            return carry

        lax.fori_loop(0, n_chunks, chunk, 0)
        plsc.subcore_barrier()
        pltpu.sync_copy(acc_m.at[pl.ds(r0, rows_pt)],
                        mp_hbm.at[cid, pl.ds(r0, rows_pt)])
        pltpu.sync_copy(acc_f.at[pl.ds(r0, rows_pt)],
                        fp_hbm.at[cid, pl.ds(r0, rows_pt)])

    return scatter_k


def _node_body(x_ref, h_ref, mp_ref, fp_ref,
               nw1a_ref, nw1b_ref, nb1_ref, nw2_ref, nb2_ref,
               x_out_ref, h_out_ref):
    tot = mp_ref[0] + mp_ref[1]
    ff = fp_ref[0] + fp_ref[1]
    cnt = jnp.maximum(ff[:, 3:4], 1.0)
    x_out_ref[...] = x_ref[...] + ff[:, 0:3] / cnt
    pre = (jnp.dot(h_ref[...], nw1a_ref[...], preferred_element_type=jnp.float32)
           + jnp.dot(tot, nw1b_ref[...], preferred_element_type=jnp.float32)
           + nb1_ref[...])
    h_out_ref[...] = jnp.dot(jax.nn.silu(pre), nw2_ref[...],
                             preferred_element_type=jnp.float32) + nb2_ref[...]


def kernel(x, h, edge_fea, edge_index, ew1, eb1, ew2, eb2,
           cw1, cb1, cw2, cb2, nw1, nb1, nw2, nb2):
    N, H = h.shape
    E = edge_index.shape[1]
    Fe = edge_fea.shape[1]
    row = edge_index[0]
    col = edge_index[1]
    xp = jnp.pad(x, ((0, 0), (0, _L - x.shape[1])))
    wa = ew1[1:1 + H]
    wb = ew1[1 + H:1 + 2 * H]
    w0 = ew1[0:1]
    wf = ew1[1 + 2 * H:]

    sds = jax.ShapeDtypeStruct
    a, b = pl.pallas_call(
        _premul_body,
        out_shape=(sds((N, H), jnp.float32), sds((N, H), jnp.float32)),
    )(h, wa, wb)

    hr, hc, xr, xc = _make_sc_gather(N, E, H)(a, b, xp, row, col)

    Eb = 2000
    msg, fr = pl.pallas_call(
        _edge_body,
        grid=(E // Eb,),
        in_specs=[
            pl.BlockSpec((Eb, H), lambda i: (i, 0)),
            pl.BlockSpec((Eb, H), lambda i: (i, 0)),
            pl.BlockSpec((Eb, _L), lambda i: (i, 0)),
            pl.BlockSpec((Eb, _L), lambda i: (i, 0)),
            pl.BlockSpec((Eb, Fe), lambda i: (i, 0)),
            pl.BlockSpec((1, H), lambda i: (0, 0)),
            pl.BlockSpec((Fe, H), lambda i: (0, 0)),
            pl.BlockSpec((1, H), lambda i: (0, 0)),
            pl.BlockSpec((H, H), lambda i: (0, 0)),
            pl.BlockSpec((1, H), lambda i: (0, 0)),
            pl.BlockSpec((H, H), lambda i: (0, 0)),
            pl.BlockSpec((1, H), lambda i: (0, 0)),
            pl.BlockSpec((H, 1), lambda i: (0, 0)),
            pl.BlockSpec((1, 1), lambda i: (0, 0)),
        ],
        out_specs=[
            pl.BlockSpec((Eb, H), lambda i: (i, 0)),
            pl.BlockSpec((Eb, _L), lambda i: (i, 0)),
        ],
        out_shape=(sds((E, H), jnp.float32), sds((E, _L), jnp.float32)),
        compiler_params=pltpu.CompilerParams(
            dimension_semantics=("arbitrary",)),
    )(hr, hc, xr, xc, edge_fea,
      w0, wf, eb1.reshape(1, H), ew2, eb2.reshape(1, H),
      cw1, cb1.reshape(1, H), cw2, cb2.reshape(1, 1))

    z_m = jnp.zeros((N, H), jnp.float32)
    z_f = jnp.zeros((N, _L), jnp.float32)
    mp, fp = _make_sc_scatter(N, E, H)(msg, fr, row, z_m, z_f)

    x_out, h_out = pl.pallas_call(
        _node_body,
        out_shape=(sds((N, x.shape[1]), jnp.float32), sds((N, H), jnp.float32)),
    )(x, h, mp, fp, nw1[:H], nw1[H:], nb1.reshape(1, H), nw2, nb2.reshape(1, H))

    return (x_out, h_out)


# speedup vs baseline: 4.4659x; 1.3223x over previous
import functools

import jax
import jax.numpy as jnp
from jax import lax
from jax.experimental import pallas as pl
from jax.experimental.pallas import tpu as pltpu
from jax.experimental.pallas import tpu_sc as plsc

_NC, _NS, _L = 2, 16, 16
_NW = _NC * _NS


def _premul_body(h_ref, wa_ref, wb_ref, a_ref, b_ref):
    h = h_ref[...]
    a_ref[...] = jnp.dot(h, wa_ref[...], preferred_element_type=jnp.float32)
    b_ref[...] = jnp.dot(h, wb_ref[...], preferred_element_type=jnp.float32)


def _make_sc_gather(N, E, H):
    ept = E // _NW
    K = 80
    n_chunks = ept // K

    mesh = plsc.VectorSubcoreMesh(
        core_axis_name="c", subcore_axis_name="s",
        num_cores=_NC, num_subcores=_NS)

    @functools.partial(
        pl.kernel,
        out_type=(
            jax.ShapeDtypeStruct((E, H), jnp.float32),
            jax.ShapeDtypeStruct((E, H), jnp.float32),
            jax.ShapeDtypeStruct((E, _L), jnp.float32),
            jax.ShapeDtypeStruct((E, _L), jnp.float32),
        ),
        mesh=mesh,
        scratch_types=[
            pltpu.VMEM((n_chunks, K), jnp.int32),
            pltpu.VMEM((n_chunks, K), jnp.int32),
            pltpu.VMEM((K, H), jnp.float32),
            pltpu.VMEM((K, H), jnp.float32),
            pltpu.VMEM((K, H), jnp.float32),
            pltpu.VMEM((K, H), jnp.float32),
            pltpu.VMEM((K, _L), jnp.float32),
            pltpu.VMEM((K, _L), jnp.float32),
            pltpu.VMEM((K, _L), jnp.float32),
            pltpu.VMEM((K, _L), jnp.float32),
            pltpu.SemaphoreType.DMA,
            pltpu.SemaphoreType.DMA,
            pltpu.SemaphoreType.DMA,
            pltpu.SemaphoreType.DMA,
        ],
        compiler_params=pltpu.CompilerParams(use_tc_tiling_on_sc=False),
    )
    def gather_k(a_hbm, b_hbm, xp_hbm, row3_hbm, col3_hbm,
                 hr_hbm, hc_hbm, xr_hbm, xc_hbm,
                 idx_r, idx_c, a0, a1, b0, b1, xr0, xr1, xc0, xc1,
                 g0, g1, w0, w1):
        wid = lax.axis_index("c") * _NS + lax.axis_index("s")
        base0 = wid * ept
        pltpu.sync_copy(row3_hbm.at[wid], idx_r)
        pltpu.sync_copy(col3_hbm.at[wid], idx_c)

        slots = ((a0, b0, xr0, xc0, g0, w0), (a1, b1, xr1, xc1, g1, w1))

        def gathers(ci, s, wait):
            ba, bb, bxr, bxc, g, _ = s
            ir = idx_r.at[ci]
            ic = idx_c.at[ci]
            for table, idx, buf in ((a_hbm, ir, ba), (b_hbm, ic, bb),
                                    (xp_hbm, ir, bxr), (xp_hbm, ic, bxc)):
                if wait:
                    pltpu.make_async_copy(table.at[idx], buf, g).wait()
                else:
                    pltpu.async_copy(table.at[idx], buf, g)

        def writebacks(ci, s, wait):
            ba, bb, bxr, bxc, _, w = s
            base = base0 + ci * K
            for buf, out in ((ba, hr_hbm), (bb, hc_hbm),
                             (bxr, xr_hbm), (bxc, xc_hbm)):
                dst = out.at[pl.ds(base, K)]
                if wait:
                    pltpu.make_async_copy(buf, dst, w).wait()
                else:
                    pltpu.async_copy(buf, dst, w)

        gathers(0, slots[0], False)
        gathers(1, slots[1], False)
        gathers(0, slots[0], True)
        writebacks(0, slots[0], False)
        gathers(1, slots[1], True)
        writebacks(1, slots[1], False)

        def pair(j, carry):
            c0 = 2 * j
            c1 = c0 + 1
            writebacks(c0 - 2, slots[0], True)
            gathers(c0, slots[0], False)
            writebacks(c1 - 2, slots[1], True)
            gathers(c1, slots[1], False)
            gathers(c0, slots[0], True)
            writebacks(c0, slots[0], False)
            gathers(c1, slots[1], True)
            writebacks(c1, slots[1], False)
            return carry

        lax.fori_loop(1, n_chunks // 2, pair, 0)

        last = n_chunks - 1
        writebacks(last - 2, slots[0], True)
        gathers(last, slots[0], False)
        gathers(last, slots[0], True)
        writebacks(last, slots[0], False)
        writebacks(last - 1, slots[1], True)
        writebacks(last, slots[0], True)

    return gather_k


def _edge_body(hr_ref, hc_ref, xr_ref, xc_ref, ef_ref,
               w0_ref, wf_ref, eb1_ref, ew2_ref, eb2_ref,
               cw1_ref, cb1_ref, cw2_ref, cb2_ref,
               msg_ref, fr_ref):
    rij = xr_ref[...] - xc_ref[...]
    sq = jnp.sum(rij * rij, axis=-1, keepdims=True)
    pre = hr_ref[...] + hc_ref[...]
    pre = pre + sq * w0_ref[...]
    pre = pre + jnp.dot(ef_ref[...], wf_ref[...],
                        preferred_element_type=jnp.float32)
    m = jax.nn.silu(pre + eb1_ref[...])
    msg = jax.nn.silu(jnp.dot(m, ew2_ref[...],
                              preferred_element_type=jnp.float32) + eb2_ref[...])
    hid = jax.nn.silu(jnp.dot(msg, cw1_ref[...],
                              preferred_element_type=jnp.float32) + cb1_ref[...])
    cm = jnp.dot(hid, cw2_ref[...],
                 preferred_element_type=jnp.float32) + cb2_ref[...]
    msg_ref[...] = msg
    lane = lax.broadcasted_iota(jnp.int32, rij.shape, 1)
    fr_ref[...] = rij * cm + jnp.where(lane == 3, 1.0, 0.0).astype(jnp.float32)


def _make_sc_scatter(N, E, H):
    ept = E // _NW
    K = 80
    n_chunks = ept // K
    rows_pt = N // _NS
    mesh = plsc.VectorSubcoreMesh(
        core_axis_name="c", subcore_axis_name="s",
        num_cores=_NC, num_subcores=_NS)

    @functools.partial(
        pl.kernel,
        out_type=(
            jax.ShapeDtypeStruct((_NC, N, H), jnp.float32),
            jax.ShapeDtypeStruct((_NC, N, _L), jnp.float32),
        ),
        mesh=mesh,
        scratch_types=[
            pltpu.VMEM((n_chunks, K), jnp.int32),
            pltpu.VMEM((K, H), jnp.float32),
            pltpu.VMEM((K, H), jnp.float32),
            pltpu.VMEM((K, _L), jnp.float32),
            pltpu.VMEM((K, _L), jnp.float32),
            pltpu.VMEM_SHARED((N, H), jnp.float32),
            pltpu.VMEM_SHARED((N, _L), jnp.float32),
            pltpu.SemaphoreType.DMA,
            pltpu.SemaphoreType.DMA,
        ],
        compiler_params=pltpu.CompilerParams(use_tc_tiling_on_sc=False),
    )
    def scatter_k(msg_hbm, fr_hbm, row3_hbm, z_m_hbm, z_f_hbm,
                  mp_hbm, fp_hbm,
                  idx_v, m0, m1, f0, f1, acc_m, acc_f, l0, l1):
        cid = lax.axis_index("c")
        sid = lax.axis_index("s")
        wid = cid * _NS + sid
        r0 = sid * rows_pt
        pltpu.sync_copy(z_m_hbm.at[pl.ds(r0, rows_pt)],
                        acc_m.at[pl.ds(r0, rows_pt)])
        pltpu.sync_copy(z_f_hbm.at[pl.ds(r0, rows_pt)],
                        acc_f.at[pl.ds(r0, rows_pt)])
        pltpu.sync_copy(row3_hbm.at[wid], idx_v)
        plsc.subcore_barrier()
        base0 = wid * ept

        slots = ((m0, f0, l0), (m1, f1, l1))

        def loads(ci, s, wait):
            bm, bf, sem = s
            base = base0 + ci * K
            for src, buf in ((msg_hbm, bm), (fr_hbm, bf)):
                if wait:
                    pltpu.make_async_copy(src.at[pl.ds(base, K)], buf, sem).wait()
                else:
                    pltpu.async_copy(src.at[pl.ds(base, K)], buf, sem)

        def scatters(ci, s):
            bm, bf, _ = s
            idx = idx_v.at[ci]
            pltpu.sync_copy(bm, acc_m.at[idx], add=True)
            pltpu.sync_copy(bf, acc_f.at[idx], add=True)

        loads(0, slots[0], False)
        loads(1, slots[1], False)

        def pair(j, carry):
            c0 = 2 * j
            c1 = c0 + 1
            loads(c0, slots[0], True)
            scatters(c0, slots[0])
            loads(c0 + 2, slots[0], False)
            loads(c1, slots[1], True)
            scatters(c1, slots[1])
            loads(c1 + 2, slots[1], False)
            return carry

        lax.fori_loop(0, n_chunks // 2 - 1, pair, 0)

        loads(n_chunks - 3, slots[0], True)
        scatters(n_chunks - 3, slots[0])
        loads(n_chunks - 1, slots[0], False)
        loads(n_chunks - 2, slots[1], True)
        scatters(n_chunks - 2, slots[1])
        loads(n_chunks - 1, slots[0], True)
        scatters(n_chunks - 1, slots[0])
        plsc.subcore_barrier()
        pltpu.sync_copy(acc_m.at[pl.ds(r0, rows_pt)],
                        mp_hbm.at[cid, pl.ds(r0, rows_pt)])
        pltpu.sync_copy(acc_f.at[pl.ds(r0, rows_pt)],
                        fp_hbm.at[cid, pl.ds(r0, rows_pt)])

    return scatter_k


def _node_body(x_ref, h_ref, mp_ref, fp_ref,
               nw1a_ref, nw1b_ref, nb1_ref, nw2_ref, nb2_ref,
               x_out_ref, h_out_ref):
    tot = mp_ref[0] + mp_ref[1]
    ff = fp_ref[0] + fp_ref[1]
    cnt = jnp.maximum(ff[:, 3:4], 1.0)
    x_out_ref[...] = x_ref[...] + ff[:, 0:3] / cnt
    pre = (jnp.dot(h_ref[...], nw1a_ref[...], preferred_element_type=jnp.float32)
           + jnp.dot(tot, nw1b_ref[...], preferred_element_type=jnp.float32)
           + nb1_ref[...])
    h_out_ref[...] = jnp.dot(jax.nn.silu(pre), nw2_ref[...],
                             preferred_element_type=jnp.float32) + nb2_ref[...]


def kernel(x, h, edge_fea, edge_index, ew1, eb1, ew2, eb2,
           cw1, cb1, cw2, cb2, nw1, nb1, nw2, nb2):
    N, H = h.shape
    E = edge_index.shape[1]
    Fe = edge_fea.shape[1]
    ept = E // _NW
    K = 80
    n_chunks = ept // K
    row = edge_index[0]
    col = edge_index[1]
    row3 = row.reshape(_NW, n_chunks, K)
    col3 = edge_index[1].reshape(_NW, n_chunks, K)
    xp = jnp.pad(x, ((0, 0), (0, _L - x.shape[1])))
    wa = ew1[1:1 + H]
    wb = ew1[1 + H:1 + 2 * H]
    w0 = ew1[0:1]
    wf = ew1[1 + 2 * H:]

    sds = jax.ShapeDtypeStruct
    a, b = pl.pallas_call(
        _premul_body,
        out_shape=(sds((N, H), jnp.float32), sds((N, H), jnp.float32)),
    )(h, wa, wb)

    hr, hc, xr, xc = _make_sc_gather(N, E, H)(a, b, xp, row3, col3)

    Eb = 2000
    msg, fr = pl.pallas_call(
        _edge_body,
        grid=(E // Eb,),
        in_specs=[
            pl.BlockSpec((Eb, H), lambda i: (i, 0)),
            pl.BlockSpec((Eb, H), lambda i: (i, 0)),
            pl.BlockSpec((Eb, _L), lambda i: (i, 0)),
            pl.BlockSpec((Eb, _L), lambda i: (i, 0)),
            pl.BlockSpec((Eb, Fe), lambda i: (i, 0)),
            pl.BlockSpec((1, H), lambda i: (0, 0)),
            pl.BlockSpec((Fe, H), lambda i: (0, 0)),
            pl.BlockSpec((1, H), lambda i: (0, 0)),
            pl.BlockSpec((H, H), lambda i: (0, 0)),
            pl.BlockSpec((1, H), lambda i: (0, 0)),
            pl.BlockSpec((H, H), lambda i: (0, 0)),
            pl.BlockSpec((1, H), lambda i: (0, 0)),
            pl.BlockSpec((H, 1), lambda i: (0, 0)),
            pl.BlockSpec((1, 1), lambda i: (0, 0)),
        ],
        out_specs=[
            pl.BlockSpec((Eb, H), lambda i: (i, 0)),
            pl.BlockSpec((Eb, _L), lambda i: (i, 0)),
        ],
        out_shape=(sds((E, H), jnp.float32), sds((E, _L), jnp.float32)),
        compiler_params=pltpu.CompilerParams(
            dimension_semantics=("arbitrary",)),
    )(hr, hc, xr, xc, edge_fea,
      w0, wf, eb1.reshape(1, H), ew2, eb2.reshape(1, H),
      cw1, cb1.reshape(1, H), cw2, cb2.reshape(1, 1))

    z_m = jnp.zeros((N, H), jnp.float32)
    z_f = jnp.zeros((N, _L), jnp.float32)
    mp, fp = _make_sc_scatter(N, E, H)(msg, fr, row3, z_m, z_f)

    x_out, h_out = pl.pallas_call(
        _node_body,
        out_shape=(sds((N, x.shape[1]), jnp.float32), sds((N, H), jnp.float32)),
    )(x, h, mp, fp, nw1[:H], nw1[H:], nb1.reshape(1, H), nw2, nb2.reshape(1, H))

    return (x_out, h_out)
